# baseline (speedup 1.0000x reference)
import functools

import jax
import jax.numpy as jnp
from jax import lax
from jax.experimental import pallas as pl
from jax.experimental.pallas import tpu as pltpu
from jax.experimental.pallas import tpu_sc as plsc

BATCH = 16384
NUM_FIELDS = 26
EMBED_DIM = 16
B_TOTAL = BATCH * NUM_FIELDS

NUM_CORES = 2
NUM_SUBCORES = 16
NUM_WORKERS = NUM_CORES * NUM_SUBCORES
B_PER_W = B_TOTAL // NUM_WORKERS
CHUNK = 1664
NCHUNK = B_PER_W // CHUNK
LANES = 16


def _emb_body(inp_hbm, offs_hbm, table_hbm, out_hbm, idx_v, offs_v, rows_v, sem):
    wid = lax.axis_index("s") * NUM_CORES + lax.axis_index("c")
    base = wid * B_PER_W

    def chunk_body(c, carry):
        cb = pl.multiple_of(base + c * CHUNK, 8)
        pltpu.sync_copy(inp_hbm.at[pl.ds(cb, CHUNK)], idx_v)
        pltpu.sync_copy(offs_hbm.at[pl.ds(cb, CHUNK)], offs_v)

        def add_body(i, carry2):
            s = pl.multiple_of(i * LANES, LANES)
            idx_v[pl.ds(s, LANES)] = idx_v[pl.ds(s, LANES)] + offs_v[pl.ds(s, LANES)]
            return carry2

        lax.fori_loop(0, CHUNK // LANES, add_body, 0)
        pltpu.async_copy(table_hbm.at[idx_v], rows_v, sem).wait()
        pltpu.sync_copy(rows_v, out_hbm.at[pl.ds(cb, CHUNK)])
        return carry

    lax.fori_loop(0, NCHUNK, chunk_body, 0)


_emb = functools.partial(
    pl.kernel,
    out_type=jax.ShapeDtypeStruct((B_TOTAL, EMBED_DIM), jnp.float32),
    mesh=plsc.VectorSubcoreMesh(core_axis_name="c", subcore_axis_name="s"),
    scratch_types=[
        pltpu.VMEM((CHUNK,), jnp.int32),
        pltpu.VMEM((CHUNK,), jnp.int32),
        pltpu.VMEM((CHUNK, EMBED_DIM), jnp.float32),
        pltpu.SemaphoreType.DMA,
    ],
    compiler_params=pltpu.CompilerParams(use_tc_tiling_on_sc=False),
)(_emb_body)


def kernel(input_x, offsets, table):
    flat = input_x.astype(jnp.int32).reshape(B_TOTAL)
    offs_full = jnp.tile(offsets.astype(jnp.int32), BATCH)
    out = _emb(flat, offs_full, table)
    return out.reshape(BATCH, NUM_FIELDS, EMBED_DIM)


# speedup vs baseline: 1.0031x; 1.0031x over previous
import functools

import jax
import jax.numpy as jnp
from jax import lax
from jax.experimental import pallas as pl
from jax.experimental.pallas import tpu as pltpu
from jax.experimental.pallas import tpu_sc as plsc

BATCH = 16384
NUM_FIELDS = 26
EMBED_DIM = 16
B_TOTAL = BATCH * NUM_FIELDS
TOTAL_ROWS = 2600000

NUM_CORES = 2
NUM_SUBCORES = 16
NUM_WORKERS = NUM_CORES * NUM_SUBCORES
B_PER_W = B_TOTAL // NUM_WORKERS
IDX_W = 128
CHUNK_ROWS = 13
CHUNK = CHUNK_ROWS * IDX_W
NCHUNK = B_PER_W // CHUNK
ROW_BASE_W = B_PER_W // IDX_W
LANES = 16


def _emb_body(inp_hbm, offs_hbm, table_hbm, out_hbm, idx_v, offs_v, rows_v, sem):
    wid = lax.axis_index("s") * NUM_CORES + lax.axis_index("c")
    row_base = wid * ROW_BASE_W

    pltpu.sync_copy(offs_hbm, offs_v)

    def chunk_body(c, carry):
        rb = pl.multiple_of(row_base + c * CHUNK_ROWS, CHUNK_ROWS)
        eb = pl.multiple_of(rb * IDX_W, CHUNK)
        pltpu.sync_copy(inp_hbm.at[pl.ds(rb, CHUNK_ROWS)], idx_v)

        def add_body(i, carry2):
            s = pl.multiple_of(i * LANES, LANES)
            for r in range(CHUNK_ROWS):
                idx_v[r, pl.ds(s, LANES)] = (
                    idx_v[r, pl.ds(s, LANES)] + offs_v[r, pl.ds(s, LANES)]
                )
            return carry2

        lax.fori_loop(0, IDX_W // LANES, add_body, 0)

        for r in range(CHUNK_ROWS):
            pltpu.async_copy(
                table_hbm.at[idx_v.at[r]],
                rows_v.at[pl.ds(r * IDX_W, IDX_W)],
                sem,
            )
        pltpu.make_async_copy(
            table_hbm.at[pl.ds(0, CHUNK)], rows_v.at[pl.ds(0, CHUNK)], sem
        ).wait()
        pltpu.sync_copy(rows_v, out_hbm.at[pl.ds(eb, CHUNK)])
        return carry

    lax.fori_loop(0, NCHUNK, chunk_body, 0)


_emb = functools.partial(
    pl.kernel,
    out_type=jax.ShapeDtypeStruct((B_TOTAL, EMBED_DIM), jnp.float32),
    mesh=plsc.VectorSubcoreMesh(core_axis_name="c", subcore_axis_name="s"),
    scratch_types=[
        pltpu.VMEM((CHUNK_ROWS, IDX_W), jnp.int32),
        pltpu.VMEM((CHUNK_ROWS, IDX_W), jnp.int32),
        pltpu.VMEM((CHUNK, EMBED_DIM), jnp.float32),
        pltpu.SemaphoreType.DMA,
    ],
    compiler_params=pltpu.CompilerParams(use_tc_tiling_on_sc=False),
)(_emb_body)


def kernel(input_x, offsets, table):
    inp2d = input_x.astype(jnp.int32).reshape(B_TOTAL // IDX_W, IDX_W)
    offs_tile = jnp.tile(offsets.astype(jnp.int32), CHUNK // NUM_FIELDS).reshape(
        CHUNK_ROWS, IDX_W
    )
    out = _emb(inp2d, offs_tile, table)
    return out.reshape(BATCH, NUM_FIELDS, EMBED_DIM)


# speedup vs baseline: 1.0039x; 1.0009x over previous
import functools

import jax
import jax.numpy as jnp
from jax import lax
from jax.experimental import pallas as pl
from jax.experimental.pallas import tpu as pltpu
from jax.experimental.pallas import tpu_sc as plsc

BATCH = 16384
NUM_FIELDS = 26
EMBED_DIM = 16
B_TOTAL = BATCH * NUM_FIELDS
NUM_CORES = 2
NUM_SUBCORES = 16
NUM_WORKERS = NUM_CORES * NUM_SUBCORES
PER_WORKER = B_TOTAL // NUM_WORKERS
CHUNK = 1664
NUM_CHUNKS = PER_WORKER // CHUNK
IDX_W = 128
STREAMS = CHUNK // IDX_W
LANES = 16
VECS = CHUNK // LANES


def _emb_body(inp_hbm, offs_hbm, table_hbm, out_hbm, idx_v, offs_v, rows_v, sem):
    wid = lax.axis_index("s") * NUM_CORES + lax.axis_index("c")
    base = pl.multiple_of(wid * PER_WORKER, CHUNK)
    pltpu.sync_copy(offs_hbm, offs_v)

    @pl.loop(0, NUM_CHUNKS)
    def _chunk(k):
        cb = pl.multiple_of(base + k * CHUNK, CHUNK)
        pltpu.sync_copy(inp_hbm.at[pl.ds(cb, CHUNK)], idx_v)
        for j in range(VECS):
            sl = pl.ds(j * LANES, LANES)
            idx_v[sl] = idx_v[sl] + offs_v[sl]
        for i in range(STREAMS):
            pltpu.async_copy(
                table_hbm.at[idx_v.at[pl.ds(i * IDX_W, IDX_W)]],
                rows_v.at[pl.ds(i * IDX_W, IDX_W)],
                sem,
            )
        pltpu.make_async_copy(table_hbm.at[pl.ds(0, CHUNK)], rows_v, sem).wait()
        pltpu.sync_copy(rows_v, out_hbm.at[pl.ds(cb, CHUNK)])


_emb = functools.partial(
    pl.kernel,
    out_type=jax.ShapeDtypeStruct((B_TOTAL, EMBED_DIM), jnp.float32),
    mesh=plsc.VectorSubcoreMesh(core_axis_name="c", subcore_axis_name="s"),
    scratch_types=[
        pltpu.VMEM((CHUNK,), jnp.int32),
        pltpu.VMEM((CHUNK,), jnp.int32),
        pltpu.VMEM((CHUNK, EMBED_DIM), jnp.float32),
        pltpu.SemaphoreType.DMA,
    ],
    compiler_params=pltpu.CompilerParams(use_tc_tiling_on_sc=False),
)(_emb_body)


def kernel(input_x, offsets, table):
    inp_flat = input_x.astype(jnp.int32).reshape(B_TOTAL)
    offs_tile = jnp.tile(offsets.astype(jnp.int32), CHUNK // NUM_FIELDS)
    out = _emb(inp_flat, offs_tile, table)
    return out.reshape(BATCH, NUM_FIELDS, EMBED_DIM)


# speedup vs baseline: 1.0051x; 1.0011x over previous
import functools

import jax
import jax.numpy as jnp
from jax import lax
from jax.experimental import pallas as pl
from jax.experimental.pallas import tpu as pltpu
from jax.experimental.pallas import tpu_sc as plsc

BATCH = 16384
NUM_FIELDS = 26
EMBED_DIM = 16
B_TOTAL = BATCH * NUM_FIELDS
NUM_CORES = 2
NUM_SUBCORES = 16
NUM_WORKERS = NUM_CORES * NUM_SUBCORES
PER_WORKER = B_TOTAL // NUM_WORKERS
CHUNK = 1664
NUM_CHUNKS = PER_WORKER // CHUNK
IDX_W = 128
STREAMS = CHUNK // IDX_W
LANES = 16
VECS = CHUNK // LANES


def _emb_body(
    inp_hbm, offs_hbm, table_hbm, out_hbm, idx_v, offs_v, rows0_v, rows1_v, gsem, wsem
):
    wid = lax.axis_index("s") * NUM_CORES + lax.axis_index("c")
    base = pl.multiple_of(wid * PER_WORKER, CHUNK)
    pltpu.sync_copy(offs_hbm, offs_v)

    for k in range(NUM_CHUNKS):
        rows_v = rows0_v if k % 2 == 0 else rows1_v
        cb = pl.multiple_of(base + k * CHUNK, CHUNK)
        pltpu.sync_copy(inp_hbm.at[pl.ds(cb, CHUNK)], idx_v)
        for j in range(VECS):
            sl = pl.ds(j * LANES, LANES)
            idx_v[sl] = idx_v[sl] + offs_v[sl]
        if k >= 2:
            pb = pl.multiple_of(base + (k - 2) * CHUNK, CHUNK)
            pltpu.make_async_copy(rows_v, out_hbm.at[pl.ds(pb, CHUNK)], wsem).wait()
        for i in range(STREAMS):
            pltpu.async_copy(
                table_hbm.at[idx_v.at[pl.ds(i * IDX_W, IDX_W)]],
                rows_v.at[pl.ds(i * IDX_W, IDX_W)],
                gsem,
            )
        pltpu.make_async_copy(table_hbm.at[pl.ds(0, CHUNK)], rows_v, gsem).wait()
        pltpu.async_copy(rows_v, out_hbm.at[pl.ds(cb, CHUNK)], wsem)

    for k in range(NUM_CHUNKS - 2, NUM_CHUNKS):
        rows_v = rows0_v if k % 2 == 0 else rows1_v
        cb = pl.multiple_of(base + k * CHUNK, CHUNK)
        pltpu.make_async_copy(rows_v, out_hbm.at[pl.ds(cb, CHUNK)], wsem).wait()


_emb = functools.partial(
    pl.kernel,
    out_type=jax.ShapeDtypeStruct((B_TOTAL, EMBED_DIM), jnp.float32),
    mesh=plsc.VectorSubcoreMesh(core_axis_name="c", subcore_axis_name="s"),
    scratch_types=[
        pltpu.VMEM((CHUNK,), jnp.int32),
        pltpu.VMEM((CHUNK,), jnp.int32),
        pltpu.VMEM((CHUNK, EMBED_DIM), jnp.float32),
        pltpu.VMEM((CHUNK, EMBED_DIM), jnp.float32),
        pltpu.SemaphoreType.DMA,
        pltpu.SemaphoreType.DMA,
    ],
    compiler_params=pltpu.CompilerParams(use_tc_tiling_on_sc=False),
)(_emb_body)


def kernel(input_x, offsets, table):
    inp_flat = input_x.astype(jnp.int32).reshape(B_TOTAL)
    offs_tile = jnp.tile(offsets.astype(jnp.int32), CHUNK // NUM_FIELDS)
    out = _emb(inp_flat, offs_tile, table)
    return out.reshape(BATCH, NUM_FIELDS, EMBED_DIM)
